# baseline (speedup 1.0000x reference)
import functools

import jax
import jax.numpy as jnp
from jax import lax
from jax.experimental import pallas as pl
from jax.experimental.pallas import tpu as pltpu
from jax.experimental.pallas import tpu_sc as plsc

D = 32
B_TOTAL = 16384 * 50
NC, NS = 2, 16
NW = NC * NS
B_PER_W = B_TOTAL // NW
IDX_MINOR = 128
GCHUNK = 2560
K = GCHUNK // IDX_MINOR
N_GROUPS = B_PER_W // GCHUNK

_mesh = plsc.VectorSubcoreMesh(core_axis_name="c", subcore_axis_name="s")


@functools.partial(
    pl.kernel,
    out_type=jax.ShapeDtypeStruct((B_TOTAL, D), jnp.float32),
    mesh=_mesh,
    scratch_types=[
        pltpu.VMEM((GCHUNK,), jnp.int32),
        pltpu.VMEM((GCHUNK, D), jnp.float32),
        pltpu.SemaphoreType.DMA,
    ],
    compiler_params=pltpu.CompilerParams(use_tc_tiling_on_sc=False),
)
def _embedding_gather(idx_hbm, table_hbm, out_hbm, idx_v, rows_v, sem):
    wid = lax.axis_index("s") * NC + lax.axis_index("c")
    base = wid * B_PER_W

    def body(g, carry):
        off = pl.multiple_of(base + g * GCHUNK, 8)
        pltpu.sync_copy(idx_hbm.at[pl.ds(off, GCHUNK)], idx_v)
        copies = [
            pltpu.async_copy(
                table_hbm.at[idx_v.at[pl.ds(j * IDX_MINOR, IDX_MINOR)]],
                rows_v.at[pl.ds(j * IDX_MINOR, IDX_MINOR)],
                sem,
            )
            for j in range(K)
        ]
        for cp in copies:
            cp.wait()
        pltpu.sync_copy(rows_v, out_hbm.at[pl.ds(off, GCHUNK)])
        return carry

    lax.fori_loop(0, N_GROUPS, body, 0)


def kernel(token_ids, weight):
    idx_flat = token_ids.reshape(B_TOTAL).astype(jnp.int32)
    out = _embedding_gather(idx_flat, weight)
    return out.reshape(*token_ids.shape, D)


# speedup vs baseline: 1.3019x; 1.3019x over previous
import functools

import jax
import jax.numpy as jnp
from jax import lax
from jax.experimental import pallas as pl
from jax.experimental.pallas import tpu as pltpu
from jax.experimental.pallas import tpu_sc as plsc

D = 32
T_TOK = 50
N_SAMP = 16384
B_TOTAL = N_SAMP * T_TOK
NC, NS = 2, 16
NW = NC * NS
S_PER_W = N_SAMP // NW
B_PER_W = S_PER_W * T_TOK
NTC = 4

_mesh = plsc.VectorSubcoreMesh(core_axis_name="c", subcore_axis_name="s")


@functools.partial(
    pl.kernel,
    out_type=jax.ShapeDtypeStruct((T_TOK, 4, 128, 8, 128), jnp.float32),
    mesh=_mesh,
    scratch_types=[
        pltpu.VMEM((B_PER_W,), jnp.int32),
        pltpu.VMEM((4, 128), jnp.int32),
        pltpu.VMEM((4, 128, D), jnp.float32),
        pltpu.VMEM((4, 4, 8, 128), jnp.float32),
        pltpu.SemaphoreType.DMA,
        pltpu.SemaphoreType.DMA,
    ],
    compiler_params=pltpu.CompilerParams(
        use_tc_tiling_on_sc=False, needs_layout_passes=False
    ),
)
def _embedding_gather(idx_hbm, table_hbm, out_hbm, slab_v, idx_v, rows_v, tile_v, gsem, osem):
    wid = lax.axis_index("s") * NC + lax.axis_index("c")
    base = pl.multiple_of(wid * B_PER_W, 8)
    pltpu.sync_copy(idx_hbm.at[pl.ds(base, B_PER_W)], slab_v)

    lane = lax.iota(jnp.int32, 16)
    lane50 = lane * T_TOK

    def body(t, carry):
        gathers = []
        for tcl in range(4):
            for v in range(8):
                srcidx = lane50 + ((tcl * 128 + v * 16) * T_TOK + t)
                vals = plsc.load_gather(slab_v, [srcidx])
                idx_v[tcl, pl.ds(v * 16, 16)] = vals
            gathers.append(
                pltpu.async_copy(
                    table_hbm.at[idx_v.at[tcl]], rows_v.at[tcl], gsem
                )
            )
        stores = []
        for tcl in range(4):
            gathers[tcl].wait()
            for f in range(D):
                for v in range(8):
                    rsel = lane + (v * 16)
                    csel = jnp.full((16,), f, jnp.int32)
                    vals = plsc.load_gather(rows_v.at[tcl], [rsel, csel])
                    tile_v[tcl, f // 8, f % 8, pl.ds(v * 16, 16)] = vals
            for trf in range(4):
                stores.append(
                    pltpu.async_copy(
                        tile_v.at[tcl, trf],
                        out_hbm.at[t, trf, wid * 4 + tcl],
                        osem,
                    )
                )
        for cp in stores:
            cp.wait()
        return carry

    lax.fori_loop(0, T_TOK, body, 0)


def kernel(token_ids, weight):
    idx_flat = token_ids.reshape(B_TOTAL).astype(jnp.int32)
    blob = _embedding_gather(idx_flat, weight)
    return blob.transpose(2, 4, 0, 1, 3).reshape(N_SAMP, T_TOK, D)


# speedup vs baseline: 1.3026x; 1.0006x over previous
import functools

import jax
import jax.numpy as jnp
from jax import lax
from jax.experimental import pallas as pl
from jax.experimental.pallas import tpu as pltpu
from jax.experimental.pallas import tpu_sc as plsc

D = 32
T_TOK = 50
N_SAMP = 16384
B_TOTAL = N_SAMP * T_TOK
NC, NS = 2, 16
NW = NC * NS
S_PER_W = N_SAMP // NW
B_PER_W = S_PER_W * T_TOK

_mesh = plsc.VectorSubcoreMesh(core_axis_name="c", subcore_axis_name="s")


@functools.partial(
    pl.kernel,
    out_type=jax.ShapeDtypeStruct((T_TOK, 4, 128, 8, 128), jnp.float32),
    mesh=_mesh,
    scratch_types=[
        pltpu.VMEM((B_PER_W,), jnp.int32),
        pltpu.VMEM((4, 128), jnp.int32),
        pltpu.VMEM((4, 128, D), jnp.float32),
        pltpu.VMEM((4, 4, 8, 128), jnp.float32),
        pltpu.SemaphoreType.DMA,
        pltpu.SemaphoreType.DMA,
    ],
    compiler_params=pltpu.CompilerParams(
        use_tc_tiling_on_sc=False, needs_layout_passes=False
    ),
)
def _embedding_gather(idx_hbm, table_hbm, out_hbm, slab_v, idx_v, rows_v, tile_v, gsem, osem):
    wid = lax.axis_index("s") * NC + lax.axis_index("c")
    base = pl.multiple_of(wid * B_PER_W, 8)
    pltpu.sync_copy(idx_hbm.at[pl.ds(base, B_PER_W)], slab_v)

    lane = lax.iota(jnp.int32, 16)
    lane50 = lane * T_TOK

    def body(t, carry):
        gathers = []
        for tcl in range(4):
            for v in range(8):
                srcidx = lane50 + ((tcl * 128 + v * 16) * T_TOK + t)
                vals = plsc.load_gather(slab_v, [srcidx])
                idx_v[tcl, pl.ds(v * 16, 16)] = vals
            gathers.append(
                pltpu.async_copy(
                    table_hbm.at[idx_v.at[tcl]], rows_v.at[tcl], gsem
                )
            )
        stores = []
        for tcl in range(4):
            gathers[tcl].wait()
            for f in range(D):
                for v in range(8):
                    rsel = lane + (v * 16)
                    csel = jnp.full((16,), f, jnp.int32)
                    vals = plsc.load_gather(rows_v.at[tcl], [rsel, csel])
                    tile_v[tcl, f // 8, f % 8, pl.ds(v * 16, 16)] = vals
            for trf in range(4):
                stores.append(
                    pltpu.async_copy(
                        tile_v.at[tcl, trf],
                        out_hbm.at[t, trf, wid * 4 + tcl],
                        osem,
                    )
                )
        for cp in stores:
            cp.wait()
        return carry

    lax.fori_loop(0, T_TOK, body, 0)


def kernel(token_ids, weight):
    idx_flat = token_ids.reshape(B_TOTAL).astype(jnp.int32)
    blob = _embedding_gather(idx_flat, weight)
    return blob.transpose(2, 4, 0, 1, 3).reshape(N_SAMP, T_TOK, D)


# speedup vs baseline: 1.5641x; 1.2007x over previous
import functools

import jax
import jax.numpy as jnp
from jax import lax
from jax.experimental import pallas as pl
from jax.experimental.pallas import tpu as pltpu
from jax.experimental.pallas import tpu_sc as plsc

D = 32
T_TOK = 50
N_SAMP = 16384
B_TOTAL = N_SAMP * T_TOK
NC, NS = 2, 16
NW = NC * NS
S_PER_W = N_SAMP // NW
B_PER_W = S_PER_W * T_TOK

_mesh = plsc.VectorSubcoreMesh(core_axis_name="c", subcore_axis_name="s")


@functools.partial(
    pl.kernel,
    out_type=jax.ShapeDtypeStruct((T_TOK, 4, 128, 1024), jnp.float32),
    mesh=_mesh,
    scratch_types=[
        pltpu.VMEM((B_PER_W,), jnp.int32),
        pltpu.VMEM((4, 128), jnp.int32),
        pltpu.VMEM((4, 128, D), jnp.float32),
        pltpu.VMEM((4, 4096), jnp.float32),
        pltpu.SemaphoreType.DMA,
        pltpu.SemaphoreType.DMA,
    ],
    compiler_params=pltpu.CompilerParams(
        use_tc_tiling_on_sc=False, needs_layout_passes=False
    ),
)
def _embedding_gather(idx_hbm, table_hbm, out_hbm, slab_v, idx_v, rows_v, tile_v, gsem, osem):
    wid = lax.axis_index("s") * NC + lax.axis_index("c")
    base = pl.multiple_of(wid * B_PER_W, 8)
    pltpu.sync_copy(idx_hbm.at[pl.ds(base, B_PER_W)], slab_v)

    lane = lax.iota(jnp.int32, 16)
    lane50 = lane * T_TOK
    lane128 = lane * 128

    def body(t, carry):
        gathers = []
        for tcl in range(4):
            for v in range(8):
                srcidx = lane50 + ((tcl * 128 + v * 16) * T_TOK + t)
                vals = plsc.load_gather(slab_v, [srcidx])
                idx_v[tcl, pl.ds(v * 16, 16)] = vals
            gathers.append(
                pltpu.async_copy(
                    table_hbm.at[idx_v.at[tcl]], rows_v.at[tcl], gsem
                )
            )
        stores = []
        for tcl in range(4):
            gathers[tcl].wait()
            for sl in range(128):
                for h in range(2):
                    vals = rows_v[tcl, sl, pl.ds(h * 16, 16)]
                    dsel = lane128 + (h * 2048 + sl)
                    plsc.store_scatter(tile_v.at[tcl], [dsel], vals)
            for trf in range(4):
                stores.append(
                    pltpu.async_copy(
                        tile_v.at[tcl, pl.ds(trf * 1024, 1024)],
                        out_hbm.at[t, trf, wid * 4 + tcl],
                        osem,
                    )
                )
        for cp in stores:
            cp.wait()
        return carry

    lax.fori_loop(0, T_TOK, body, 0)


def kernel(token_ids, weight):
    idx_flat = token_ids.reshape(B_TOTAL).astype(jnp.int32)
    blob = _embedding_gather(idx_flat, weight)
    blob5 = blob.reshape(T_TOK, 4, 128, 8, 128)
    return blob5.transpose(2, 4, 0, 1, 3).reshape(N_SAMP, T_TOK, D)
